# baseline (speedup 1.0000x reference)
import functools

import jax
import jax.numpy as jnp
from jax import lax
from jax.experimental import pallas as pl
from jax.experimental.pallas import tpu as pltpu
from jax.experimental.pallas import tpu_sc as plsc

LANES = 16
NUM_WORKERS = 32
FPB = 4
UNROLL = 8


def _make_scorer(n_nodes, d_feat, e):
    align = 128
    group = NUM_WORKERS * align
    per_big = ((e + group - 1) // group) * align
    last_sz = e - (NUM_WORKERS - 1) * per_big
    assert 0 < last_sz <= per_big and last_sz % LANES == 0
    big_groups = per_big // LANES
    last_groups = last_sz // LANES
    n_blk = (d_feat // 2) // FPB
    n_super = n_blk // 2

    @functools.partial(
        pl.kernel,
        mesh=plsc.VectorSubcoreMesh(core_axis_name="c", subcore_axis_name="s"),
        out_type=jax.ShapeDtypeStruct((e,), jnp.float32),
        compiler_params=pltpu.CompilerParams(needs_layout_passes=False),
        scratch_types=[
            pltpu.VMEM((per_big,), jnp.int32),
            pltpu.VMEM((per_big,), jnp.int32),
            [pltpu.VMEM((FPB, n_nodes), jnp.int32) for _ in range(2)],
            pltpu.VMEM((per_big,), jnp.float32),
            pltpu.VMEM((1, n_nodes), jnp.float32),
            [pltpu.SemaphoreType.DMA for _ in range(2)],
        ],
    )
    def scorer(ht_hbm, edge_hbm, bias_hbm, out_hbm,
               idx_s, idx_d, cols, acc_v, bias_v, sem):
        wid = lax.axis_index("s") * 2 + lax.axis_index("c")
        tile_base = wid * per_big
        is_last = wid == NUM_WORKERS - 1
        n_groups = jnp.where(is_last, last_groups, big_groups)
        pltpu.sync_copy(bias_hbm, bias_v)

        @pl.when(is_last)
        def _():
            pltpu.sync_copy(edge_hbm.at[0].at[pl.ds(tile_base, last_sz)],
                            idx_s.at[pl.ds(0, last_sz)])
            pltpu.sync_copy(edge_hbm.at[1].at[pl.ds(tile_base, last_sz)],
                            idx_d.at[pl.ds(0, last_sz)])

        @pl.when(jnp.logical_not(is_last))
        def _():
            pltpu.sync_copy(edge_hbm.at[0].at[pl.ds(tile_base, per_big)], idx_s)
            pltpu.sync_copy(edge_hbm.at[1].at[pl.ds(tile_base, per_big)], idx_d)

        zeros16 = jnp.zeros((LANES,), jnp.int32)

        @plsc.parallel_loop(0, n_groups, unroll=UNROLL)
        def init_body(g):
            is16 = idx_s[pl.ds(g * LANES, LANES)]
            id16 = idx_d[pl.ds(g * LANES, LANES)]
            acc_v[pl.ds(g * LANES, LANES)] = (
                plsc.load_gather(bias_v, [zeros16, is16])
                + plsc.load_gather(bias_v, [zeros16, id16]))

        def fire(blk, b):
            pltpu.async_copy(ht_hbm.at[blk], cols[b], sem[b])

        def compute(b):
            col = cols[b]

            @plsc.parallel_loop(0, n_groups, unroll=UNROLL)
            def group_body(g):
                is16 = idx_s[pl.ds(g * LANES, LANES)]
                id16 = idx_d[pl.ds(g * LANES, LANES)]
                acc = acc_v[pl.ds(g * LANES, LANES)]
                for p in range(FPB):
                    p16 = zeros16 + p
                    ws = plsc.load_gather(col, [p16, is16])
                    wd = plsc.load_gather(col, [p16, id16])
                    a_s, b_s = plsc.unpack(
                        plsc.bitcast(ws, jnp.bfloat16),
                        format=plsc.PackFormat.INTERLEAVED)
                    a_d, b_d = plsc.unpack(
                        plsc.bitcast(wd, jnp.bfloat16),
                        format=plsc.PackFormat.INTERLEAVED)
                    acc = acc + a_s * a_d + b_s * b_d
                acc_v[pl.ds(g * LANES, LANES)] = acc

        fire(0, 0)

        def super_body(s, _):
            for b in range(2):
                blk = 2 * s + b
                pltpu.make_async_copy(ht_hbm.at[0], cols[b], sem[b]).wait()

                @pl.when(blk + 1 < n_blk)
                def _():
                    fire(blk + 1, 1 - b)

                compute(b)
            return 0

        lax.fori_loop(0, n_super, super_body, 0)

        @pl.when(is_last)
        def _():
            pltpu.sync_copy(acc_v.at[pl.ds(0, last_sz)],
                            out_hbm.at[pl.ds(tile_base, last_sz)])

        @pl.when(jnp.logical_not(is_last))
        def _():
            pltpu.sync_copy(acc_v, out_hbm.at[pl.ds(tile_base, per_big)])

    return scorer


def kernel(h, edge_index, bias):
    n_nodes, d_feat = h.shape
    e = edge_index.shape[1]

    hd = d_feat // 2
    bits = jax.lax.bitcast_convert_type(h.astype(jnp.bfloat16).T, jnp.uint16)
    words = (bits[hd:].astype(jnp.uint32) << 16) | bits[:hd].astype(jnp.uint32)
    ht = jax.lax.bitcast_convert_type(words, jnp.int32).reshape(
        hd // FPB, FPB, n_nodes)

    scorer = _make_scorer(n_nodes, d_feat, e)
    return scorer(ht, edge_index.astype(jnp.int32),
                  bias.reshape(1, n_nodes).astype(jnp.float32))


# speedup vs baseline: 1.0405x; 1.0405x over previous
import functools

import jax
import jax.numpy as jnp
from jax import lax
from jax.experimental import pallas as pl
from jax.experimental.pallas import tpu as pltpu
from jax.experimental.pallas import tpu_sc as plsc

LANES = 16
NUM_WORKERS = 32
FPB = 4
UNROLL = 4


def _make_scorer(n_nodes, d_feat, e):
    align = 128
    group = NUM_WORKERS * align
    per_big = ((e + group - 1) // group) * align
    last_sz = e - (NUM_WORKERS - 1) * per_big
    assert 0 < last_sz <= per_big and last_sz % LANES == 0
    big_groups = per_big // LANES
    last_groups = last_sz // LANES
    n_blk = (d_feat // 2) // FPB
    n_super = n_blk // 2

    @functools.partial(
        pl.kernel,
        mesh=plsc.VectorSubcoreMesh(core_axis_name="c", subcore_axis_name="s"),
        out_type=jax.ShapeDtypeStruct((e,), jnp.float32),
        compiler_params=pltpu.CompilerParams(needs_layout_passes=False),
        scratch_types=[
            pltpu.VMEM((per_big,), jnp.int32),
            pltpu.VMEM((per_big,), jnp.int32),
            [pltpu.VMEM((FPB, n_nodes), jnp.int32) for _ in range(2)],
            pltpu.VMEM((per_big,), jnp.float32),
            pltpu.VMEM((1, n_nodes), jnp.float32),
            [pltpu.SemaphoreType.DMA for _ in range(2)],
        ],
    )
    def scorer(ht_hbm, edge_hbm, bias_hbm, out_hbm,
               idx_s, idx_d, cols, acc_v, bias_v, sem):
        wid = lax.axis_index("s") * 2 + lax.axis_index("c")
        tile_base = wid * per_big
        is_last = wid == NUM_WORKERS - 1
        n_groups = jnp.where(is_last, last_groups, big_groups)
        pltpu.sync_copy(bias_hbm, bias_v)

        @pl.when(is_last)
        def _():
            pltpu.sync_copy(edge_hbm.at[0].at[pl.ds(tile_base, last_sz)],
                            idx_s.at[pl.ds(0, last_sz)])
            pltpu.sync_copy(edge_hbm.at[1].at[pl.ds(tile_base, last_sz)],
                            idx_d.at[pl.ds(0, last_sz)])

        @pl.when(jnp.logical_not(is_last))
        def _():
            pltpu.sync_copy(edge_hbm.at[0].at[pl.ds(tile_base, per_big)], idx_s)
            pltpu.sync_copy(edge_hbm.at[1].at[pl.ds(tile_base, per_big)], idx_d)

        zeros16 = jnp.zeros((LANES,), jnp.int32)

        @plsc.parallel_loop(0, n_groups, unroll=UNROLL)
        def init_body(g):
            is16 = idx_s[pl.ds(g * LANES, LANES)]
            id16 = idx_d[pl.ds(g * LANES, LANES)]
            acc_v[pl.ds(g * LANES, LANES)] = (
                plsc.load_gather(bias_v, [zeros16, is16])
                + plsc.load_gather(bias_v, [zeros16, id16]))

        def fire(blk, b):
            pltpu.async_copy(ht_hbm.at[blk], cols[b], sem[b])

        def compute(b):
            col = cols[b]

            @plsc.parallel_loop(0, n_groups, unroll=UNROLL)
            def group_body(g):
                is16 = idx_s[pl.ds(g * LANES, LANES)]
                id16 = idx_d[pl.ds(g * LANES, LANES)]
                acc = acc_v[pl.ds(g * LANES, LANES)]
                for p in range(FPB):
                    p16 = zeros16 + p
                    ws = plsc.load_gather(col, [p16, is16])
                    wd = plsc.load_gather(col, [p16, id16])
                    a_s, b_s = plsc.unpack(
                        plsc.bitcast(ws, jnp.bfloat16),
                        format=plsc.PackFormat.INTERLEAVED)
                    a_d, b_d = plsc.unpack(
                        plsc.bitcast(wd, jnp.bfloat16),
                        format=plsc.PackFormat.INTERLEAVED)
                    acc = acc + a_s * a_d + b_s * b_d
                acc_v[pl.ds(g * LANES, LANES)] = acc

        fire(0, 0)

        def super_body(s, _):
            for b in range(2):
                blk = 2 * s + b
                pltpu.make_async_copy(ht_hbm.at[0], cols[b], sem[b]).wait()

                @pl.when(blk + 1 < n_blk)
                def _():
                    fire(blk + 1, 1 - b)

                compute(b)
            return 0

        lax.fori_loop(0, n_super, super_body, 0)

        @pl.when(is_last)
        def _():
            pltpu.sync_copy(acc_v.at[pl.ds(0, last_sz)],
                            out_hbm.at[pl.ds(tile_base, last_sz)])

        @pl.when(jnp.logical_not(is_last))
        def _():
            pltpu.sync_copy(acc_v, out_hbm.at[pl.ds(tile_base, per_big)])

    return scorer


def kernel(h, edge_index, bias):
    n_nodes, d_feat = h.shape
    e = edge_index.shape[1]

    hd = d_feat // 2
    bits = jax.lax.bitcast_convert_type(h.astype(jnp.bfloat16).T, jnp.uint16)
    words = (bits[hd:].astype(jnp.uint32) << 16) | bits[:hd].astype(jnp.uint32)
    ht = jax.lax.bitcast_convert_type(words, jnp.int32).reshape(
        hd // FPB, FPB, n_nodes)

    scorer = _make_scorer(n_nodes, d_feat, e)
    return scorer(ht, edge_index.astype(jnp.int32),
                  bias.reshape(1, n_nodes).astype(jnp.float32))
